# baseline (speedup 1.0000x reference)
import functools

import jax
import jax.numpy as jnp
from jax import lax
from jax.experimental import pallas as pl
from jax.experimental.pallas import tpu as pltpu
from jax.experimental.pallas import tpu_sc as plsc

N_NODES = 10000
N_EDGES = 320000
D_MODEL = 128
N_HEADS = 8
D_HEAD = D_MODEL // N_HEADS

NUM_SC = 2
NUM_TILES = 16
D_HALF = 64
N_PASS = 2
CHUNK = 64
NCH = 320
IGRP = 16
NBUF = 4
E_PAD = NUM_TILES * NCH * CHUNK
ACC_ROWS = 10240
ROW_BLK = 1000


def _tc_precompute(x_ref, wq_ref, a1_ref, bq_ref, out_ref):
    q = jnp.dot(x_ref[...], wq_ref[...], preferred_element_type=jnp.float32)
    q = q + bq_ref[...]
    u = jnp.dot(q, a1_ref[...], preferred_element_type=jnp.float32)
    e = jnp.exp(u)
    p = q * e
    out_ref[0, 0] = e[:, :D_HALF]
    out_ref[0, 1] = e[:, D_HALF:]
    out_ref[1, 0] = p[:, :D_HALF]
    out_ref[1, 1] = p[:, D_HALF:]


def _tc_finalize(x_ref, st_ref, out_ref):
    s = jnp.concatenate((st_ref[0, 0], st_ref[0, 1]), axis=-1)
    t = jnp.concatenate((st_ref[1, 0], st_ref[1, 1]), axis=-1)
    agg = jnp.where(s > 0, t / jnp.where(s > 0, s, 1.0), 0.0)
    out_ref[...] = x_ref[...] + jnp.maximum(agg, 0.0)


def _sc_segment_sum(tab4, sidx_hbm, ridx_hbm, zeros_hbm, out,
                    tabsh, acc, sidx, ridx, g0, g1, g2, g3,
                    gs0, gs1, gs2, gs3, ss0, ss1, ss2, ss3):
    cid = lax.axis_index("c")
    sid = lax.axis_index("s")
    rows_per_tile = ACC_ROWS // NUM_TILES

    gbufs = (g0, g1, g2, g3)
    gsems = (gs0, gs1, gs2, gs3)
    ssems = (ss0, ss1, ss2, ss3)

    for p in range(N_PASS):
        pltpu.sync_copy(
            tab4.at[cid, p, pl.ds(sid * rows_per_tile, rows_per_tile)],
            tabsh.at[pl.ds(sid * rows_per_tile, rows_per_tile)])
        pltpu.sync_copy(
            zeros_hbm.at[pl.ds(sid * rows_per_tile, rows_per_tile)],
            acc.at[pl.ds(sid * rows_per_tile, rows_per_tile)])
        plsc.subcore_barrier()

        def group(g, carry):
            base = sid * NCH + g * IGRP
            pltpu.sync_copy(sidx_hbm.at[pl.ds(base, IGRP)], sidx)
            pltpu.sync_copy(ridx_hbm.at[pl.ds(base, IGRP)], ridx)

            gather_d = [None] * NBUF
            scatter_d = [None] * NBUF

            def gather(j):
                return pltpu.async_copy(
                    tabsh.at[sidx.at[j]], gbufs[j % NBUF], gsems[j % NBUF])

            gather_d[0] = gather(0)
            gather_d[1] = gather(1)
            for j in range(IGRP):
                b = j % NBUF
                gather_d[b].wait()
                scatter_d[b] = pltpu.async_copy(
                    gbufs[b], acc.at[ridx.at[j]], ssems[b], add=True)
                if j + 2 < IGRP:
                    nb = (j + 2) % NBUF
                    if j >= 2:
                        scatter_d[nb].wait()
                    gather_d[nb] = gather(j + 2)
            for t in range(IGRP - NBUF, IGRP):
                scatter_d[t % NBUF].wait()
            return carry

        lax.fori_loop(0, NCH // IGRP, group, 0)
        plsc.subcore_barrier()
        pltpu.sync_copy(
            acc.at[pl.ds(sid * rows_per_tile, rows_per_tile)],
            out.at[cid, p, pl.ds(sid * rows_per_tile, rows_per_tile)])
        plsc.subcore_barrier()


def kernel(nodes, edge_index, Wq, bq, Wa, ba):
    del ba
    Wq_all = jnp.transpose(Wq, (1, 0, 2)).reshape(D_MODEL, D_MODEL)
    bq_flat = bq.reshape(1, D_MODEL)
    A1bd = jax.scipy.linalg.block_diag(
        *[Wa[i, :D_HEAD] for i in range(N_HEADS)])

    grid = N_NODES // ROW_BLK
    tab4 = pl.pallas_call(
        _tc_precompute,
        grid=(grid,),
        in_specs=[
            pl.BlockSpec((ROW_BLK, D_MODEL), lambda i: (i, 0)),
            pl.BlockSpec((D_MODEL, D_MODEL), lambda i: (0, 0)),
            pl.BlockSpec((D_MODEL, D_MODEL), lambda i: (0, 0)),
            pl.BlockSpec((1, D_MODEL), lambda i: (0, 0)),
        ],
        out_specs=pl.BlockSpec((2, N_PASS, ROW_BLK, D_HALF),
                               lambda i: (0, 0, i, 0)),
        out_shape=jax.ShapeDtypeStruct((2, N_PASS, ACC_ROWS, D_HALF),
                                       jnp.float32),
    )(nodes, Wq_all, A1bd, bq_flat)

    senders = edge_index[0]
    receivers = edge_index[1]
    pad = E_PAD - N_EDGES
    s_pad = jnp.concatenate(
        [senders, jnp.zeros((pad,), jnp.int32)]).reshape(NUM_TILES * NCH, CHUNK)
    r_pad = jnp.concatenate(
        [receivers, jnp.full((pad,), N_NODES, jnp.int32)]
    ).reshape(NUM_TILES * NCH, CHUNK)
    zeros = jnp.zeros((ACC_ROWS, D_HALF), jnp.float32)

    mesh = plsc.VectorSubcoreMesh(core_axis_name="c", subcore_axis_name="s")
    st = pl.kernel(
        _sc_segment_sum,
        out_type=jax.ShapeDtypeStruct((2, N_PASS, ACC_ROWS, D_HALF),
                                      jnp.float32),
        mesh=mesh,
        scratch_types=[
            pltpu.VMEM_SHARED((ACC_ROWS, D_HALF), jnp.float32),
            pltpu.VMEM_SHARED((ACC_ROWS, D_HALF), jnp.float32),
            pltpu.VMEM((IGRP, CHUNK), jnp.int32),
            pltpu.VMEM((IGRP, CHUNK), jnp.int32),
            pltpu.VMEM((CHUNK, D_HALF), jnp.float32),
            pltpu.VMEM((CHUNK, D_HALF), jnp.float32),
            pltpu.VMEM((CHUNK, D_HALF), jnp.float32),
            pltpu.VMEM((CHUNK, D_HALF), jnp.float32),
        ] + [pltpu.SemaphoreType.DMA] * 8,
    )(tab4, s_pad, r_pad, zeros)

    out = pl.pallas_call(
        _tc_finalize,
        grid=(grid,),
        in_specs=[
            pl.BlockSpec((ROW_BLK, D_MODEL), lambda i: (i, 0)),
            pl.BlockSpec((2, N_PASS, ROW_BLK, D_HALF), lambda i: (0, 0, i, 0)),
        ],
        out_specs=pl.BlockSpec((ROW_BLK, D_MODEL), lambda i: (i, 0)),
        out_shape=jax.ShapeDtypeStruct((N_NODES, D_MODEL), jnp.float32),
    )(nodes, st)
    return out
